# baseline (speedup 1.0000x reference)
import jax
import jax.numpy as jnp
from jax.experimental import pallas as pl
from jax.experimental.pallas import tpu as pltpu

NUM_CLASSES = 80
NL = 21888
PPAD = 21888
LG = 128


def _assign_kernel(pt_ref, gtr_ref, gtt_ref, labr_ref, cls_ref, reg_ref):
    f32 = jnp.float32
    pt = pt_ref[...]
    px = pt[0:1, :]
    py = pt[1:2, :]
    rlo = pt[2:3, :]
    rhi = pt[3:4, :]
    glo = pt[4:5, :]
    ghi = pt[5:6, :]
    s2 = pt[6:7, :] * 0.5

    gbb = gtr_ref[...]
    gx = gbb[:, 0:1]
    gy = gbb[:, 1:2]
    gw = gbb[:, 2:3]
    gh = gbb[:, 3:4]
    gxe = gx + gw - 1.0
    gye = gy + gh - 1.0
    cx = gx + gw / 2.0
    cy = gy + gh / 2.0
    meas = jnp.maximum(gw, gh)

    d1 = px - gx
    d2 = py - gy
    d3 = gxe - px
    d4 = gye - py
    hit = jnp.minimum(jnp.minimum(d1, d2), jnp.minimum(d3, d4)) >= 0.0
    q = (jnp.maximum(jnp.minimum(d1, d3), 0.0) /
         jnp.maximum(jnp.maximum(d1, d3), 0.01)) * \
        (jnp.maximum(jnp.minimum(d2, d4), 0.0) /
         jnp.maximum(jnp.maximum(d2, d4), 0.01))
    q = jnp.sqrt(q)

    inside_core = (px >= cx - s2) & (px <= cx + s2) & \
                  (py >= cy - s2) & (py <= cy + s2) & hit
    q = jnp.where(inside_core, 1.0, q)

    ghs = (rlo <= meas) & (meas <= rhi)
    green = ghs & hit
    gray = (glo <= meas) & (meas <= ghi) & ~ghs & hit
    q = q * green.astype(f32)

    G = LG // 2
    qA = q[0:G, :]
    qB = q[G:LG, :]
    maxA = jnp.max(qA, axis=0, keepdims=True)
    maxB = jnp.max(qB, axis=0, keepdims=True)
    gidx = jax.lax.broadcasted_iota(jnp.int32, (G, NL), 0)
    candA = jnp.where(qA == maxA, gidx, G)
    candB = jnp.where(qB == maxB, gidx, G)
    midxA = jnp.min(candA, axis=0, keepdims=True)
    midxB = jnp.min(candB, axis=0, keepdims=True)
    sel = jnp.concatenate(
        [(gidx == midxA), (gidx == midxB)], axis=0).astype(f32)

    gbr = gtt_ref[...]
    gx_r = gbr[0:1, :]
    gy_r = gbr[1:2, :]
    gxe_r = gx_r + gbr[2:3, :] - 1.0
    gye_r = gy_r + gbr[3:4, :] - 1.0
    lab_r = labr_ref[0].astype(f32)
    half_r = jax.lax.broadcasted_iota(jnp.int32, (1, LG), 1) >= G
    m5 = jnp.concatenate([gx_r, gy_r, gxe_r, gye_r, lab_r], axis=0)
    z3 = jnp.zeros((3, LG), f32)
    gm = jnp.concatenate(
        [jnp.where(half_r, 0.0, m5), z3, jnp.where(half_r, m5, 0.0), z3],
        axis=0)
    target_r = labr_ref[0] + jnp.where(half_r, 128, 0)
    oh = (jax.lax.broadcasted_iota(jnp.int32, (256, LG), 0) ==
          target_r).astype(f32)

    t = jnp.dot(gm, sel, preferred_element_type=f32,
                precision=jax.lax.Precision.HIGHEST)
    gcv = jnp.dot(oh, gray.astype(f32), preferred_element_type=f32)

    ciota = jax.lax.broadcasted_iota(jnp.int32, (NUM_CLASSES, NL), 0)

    posA = maxA > 0.0
    valA = jnp.where(posA, maxA, 0.0)
    matchedA = t[4:5, :].astype(jnp.int32)
    clsA = jnp.where(ciota == matchedA, valA, 0.0)
    grayA = gcv[0:NUM_CLASSES, :] > 0.0
    clsA = jnp.where(grayA & (clsA == 0.0), -1.0, clsA)
    cls_ref[0] = clsA
    regA = jnp.concatenate(
        [px - t[0:1, :], py - t[1:2, :], t[2:3, :] - px, t[3:4, :] - py],
        axis=0) * posA.astype(f32)
    reg_ref[0] = regA

    posB = maxB > 0.0
    valB = jnp.where(posB, maxB, 0.0)
    matchedB = t[12:13, :].astype(jnp.int32)
    clsB = jnp.where(ciota == matchedB, valB, 0.0)
    grayB = gcv[128:128 + NUM_CLASSES, :] > 0.0
    clsB = jnp.where(grayB & (clsB == 0.0), -1.0, clsB)
    cls_ref[1] = clsB
    regB = jnp.concatenate(
        [px - t[8:9, :], py - t[9:10, :], t[10:11, :] - px, t[11:12, :] - py],
        axis=0) * posB.astype(f32)
    reg_ref[1] = regB


def kernel(points, reg_ranges, gray_ranges, strides, gt_bboxes, gt_labels):
    P = points.shape[0]
    B, G, _ = gt_bboxes.shape
    assert PPAD % NL == 0 and B % 2 == 0 and 2 * G == LG
    npb = PPAD // NL
    B2 = B // 2

    ptall = jnp.concatenate(
        [points, reg_ranges, gray_ranges, strides.reshape(P, 1)], axis=1)
    pt_t = jnp.pad(ptall, ((0, PPAD - P), (0, 0))).T
    gt_raw = gt_bboxes.reshape(B * G, 4)
    gt_t = gt_bboxes.reshape(B * G, 4).T
    lab_row = gt_labels.reshape(B2, 1, LG)

    cls_t, reg_t = pl.pallas_call(
        _assign_kernel,
        grid=(B2, npb),
        in_specs=[
            pl.BlockSpec((7, NL), lambda b, i: (0, i)),
            pl.BlockSpec((LG, 4), lambda b, i: (b, 0)),
            pl.BlockSpec((4, LG), lambda b, i: (0, b)),
            pl.BlockSpec((1, 1, LG), lambda b, i: (b, 0, 0)),
        ],
        out_specs=[
            pl.BlockSpec((2, NUM_CLASSES, NL), lambda b, i: (b, 0, i)),
            pl.BlockSpec((2, 4, NL), lambda b, i: (b, 0, i)),
        ],
        out_shape=[
            jax.ShapeDtypeStruct((B, NUM_CLASSES, PPAD), jnp.float32),
            jax.ShapeDtypeStruct((B, 4, PPAD), jnp.float32),
        ],
        compiler_params=pltpu.CompilerParams(
            dimension_semantics=("parallel", "arbitrary"),
        ),
    )(pt_t, gt_raw, gt_t, lab_row)
    cls = jnp.transpose(cls_t[:, :, :P], (0, 2, 1))
    reg = jnp.transpose(reg_t[:, :, :P], (0, 2, 1))
    return cls, reg


# speedup vs baseline: 1.1294x; 1.1294x over previous
import jax
import jax.numpy as jnp
from jax.experimental import pallas as pl
from jax.experimental.pallas import tpu as pltpu

NUM_CLASSES = 80
NL = 7296
PPAD = 21888
LG = 128


def _assign_kernel(pt_ref, gtr_ref, gtt_ref, labr_ref, cls_ref, reg_ref):
    f32 = jnp.float32
    pt = pt_ref[...]
    px = pt[0:1, :]
    py = pt[1:2, :]
    rlo = pt[2:3, :]
    rhi = pt[3:4, :]
    glo = pt[4:5, :]
    ghi = pt[5:6, :]
    s2 = pt[6:7, :] * 0.5

    gbb = gtr_ref[...]
    gx = gbb[:, 0:1]
    gy = gbb[:, 1:2]
    gw = gbb[:, 2:3]
    gh = gbb[:, 3:4]
    gxe = gx + gw - 1.0
    gye = gy + gh - 1.0
    cx = gx + gw / 2.0
    cy = gy + gh / 2.0
    meas = jnp.maximum(gw, gh)

    d1 = px - gx
    d2 = py - gy
    d3 = gxe - px
    d4 = gye - py
    hit = jnp.minimum(jnp.minimum(d1, d2), jnp.minimum(d3, d4)) >= 0.0
    q = (jnp.maximum(jnp.minimum(d1, d3), 0.0) /
         jnp.maximum(jnp.maximum(d1, d3), 0.01)) * \
        (jnp.maximum(jnp.minimum(d2, d4), 0.0) /
         jnp.maximum(jnp.maximum(d2, d4), 0.01))
    q = jnp.sqrt(q)

    inside_core = (px >= cx - s2) & (px <= cx + s2) & \
                  (py >= cy - s2) & (py <= cy + s2) & hit
    q = jnp.where(inside_core, 1.0, q)

    ghs = (rlo <= meas) & (meas <= rhi)
    green = ghs & hit
    gray = (glo <= meas) & (meas <= ghi) & ~ghs & hit
    q = q * green.astype(f32)

    G = LG // 2
    qA = q[0:G, :]
    qB = q[G:LG, :]
    maxA = jnp.max(qA, axis=0, keepdims=True)
    maxB = jnp.max(qB, axis=0, keepdims=True)
    gidx = jax.lax.broadcasted_iota(jnp.int32, (G, NL), 0)
    candA = jnp.where(qA == maxA, gidx, G)
    candB = jnp.where(qB == maxB, gidx, G)
    midxA = jnp.min(candA, axis=0, keepdims=True)
    midxB = jnp.min(candB, axis=0, keepdims=True)
    sel = jnp.concatenate(
        [(gidx == midxA), (gidx == midxB)], axis=0).astype(f32)

    gbr = gtt_ref[...]
    gx_r = gbr[0:1, :]
    gy_r = gbr[1:2, :]
    gxe_r = gx_r + gbr[2:3, :] - 1.0
    gye_r = gy_r + gbr[3:4, :] - 1.0
    lab_r = labr_ref[0].astype(f32)
    half_r = jax.lax.broadcasted_iota(jnp.int32, (1, LG), 1) >= G
    m5 = jnp.concatenate([gx_r, gy_r, gxe_r, gye_r, lab_r], axis=0)
    z3 = jnp.zeros((3, LG), f32)
    gm = jnp.concatenate(
        [jnp.where(half_r, 0.0, m5), z3, jnp.where(half_r, m5, 0.0), z3],
        axis=0)
    target_r = labr_ref[0] + jnp.where(half_r, 128, 0)
    oh = (jax.lax.broadcasted_iota(jnp.int32, (256, LG), 0) ==
          target_r).astype(f32)

    t = jnp.dot(gm, sel, preferred_element_type=f32,
                precision=jax.lax.Precision.HIGHEST)
    gcv = jnp.dot(oh, gray.astype(f32), preferred_element_type=f32)

    ciota = jax.lax.broadcasted_iota(jnp.int32, (NUM_CLASSES, NL), 0)

    posA = maxA > 0.0
    valA = jnp.where(posA, maxA, 0.0)
    matchedA = t[4:5, :].astype(jnp.int32)
    clsA = jnp.where(ciota == matchedA, valA, 0.0)
    grayA = gcv[0:NUM_CLASSES, :] > 0.0
    clsA = jnp.where(grayA & (clsA == 0.0), -1.0, clsA)
    cls_ref[0] = clsA
    regA = jnp.concatenate(
        [px - t[0:1, :], py - t[1:2, :], t[2:3, :] - px, t[3:4, :] - py],
        axis=0) * posA.astype(f32)
    reg_ref[0] = regA

    posB = maxB > 0.0
    valB = jnp.where(posB, maxB, 0.0)
    matchedB = t[12:13, :].astype(jnp.int32)
    clsB = jnp.where(ciota == matchedB, valB, 0.0)
    grayB = gcv[128:128 + NUM_CLASSES, :] > 0.0
    clsB = jnp.where(grayB & (clsB == 0.0), -1.0, clsB)
    cls_ref[1] = clsB
    regB = jnp.concatenate(
        [px - t[8:9, :], py - t[9:10, :], t[10:11, :] - px, t[11:12, :] - py],
        axis=0) * posB.astype(f32)
    reg_ref[1] = regB


def kernel(points, reg_ranges, gray_ranges, strides, gt_bboxes, gt_labels):
    P = points.shape[0]
    B, G, _ = gt_bboxes.shape
    assert PPAD % NL == 0 and B % 2 == 0 and 2 * G == LG
    npb = PPAD // NL
    B2 = B // 2

    ptall = jnp.concatenate(
        [points, reg_ranges, gray_ranges, strides.reshape(P, 1)], axis=1)
    pt_t = jnp.pad(ptall, ((0, PPAD - P), (0, 0))).T
    gt_raw = gt_bboxes.reshape(B * G, 4)
    gt_t = gt_bboxes.reshape(B * G, 4).T
    lab_row = gt_labels.reshape(B2, 1, LG)

    cls_t, reg_t = pl.pallas_call(
        _assign_kernel,
        grid=(B2, npb),
        in_specs=[
            pl.BlockSpec((7, NL), lambda b, i: (0, i)),
            pl.BlockSpec((LG, 4), lambda b, i: (b, 0)),
            pl.BlockSpec((4, LG), lambda b, i: (0, b)),
            pl.BlockSpec((1, 1, LG), lambda b, i: (b, 0, 0)),
        ],
        out_specs=[
            pl.BlockSpec((2, NUM_CLASSES, NL), lambda b, i: (b, 0, i)),
            pl.BlockSpec((2, 4, NL), lambda b, i: (b, 0, i)),
        ],
        out_shape=[
            jax.ShapeDtypeStruct((B, NUM_CLASSES, PPAD), jnp.float32),
            jax.ShapeDtypeStruct((B, 4, PPAD), jnp.float32),
        ],
        compiler_params=pltpu.CompilerParams(
            dimension_semantics=("parallel", "arbitrary"),
        ),
    )(pt_t, gt_raw, gt_t, lab_row)
    cls = jnp.transpose(cls_t[:, :, :P], (0, 2, 1))
    reg = jnp.transpose(reg_t[:, :, :P], (0, 2, 1))
    return cls, reg


# speedup vs baseline: 1.1308x; 1.0013x over previous
import jax
import jax.numpy as jnp
from jax.experimental import pallas as pl
from jax.experimental.pallas import tpu as pltpu

NUM_CLASSES = 80
NL = 7296
PPAD = 21888
LG = 128


def _assign_kernel(pt_ref, gtr_ref, gtt_ref, labr_ref, cls_ref, reg_ref):
    f32 = jnp.float32
    pt = pt_ref[...]
    px = pt[0:1, :]
    py = pt[1:2, :]
    rlo = pt[2:3, :]
    rhi = pt[3:4, :]
    glo = pt[4:5, :]
    ghi = pt[5:6, :]
    s2 = pt[6:7, :] * 0.5

    gbb = gtr_ref[...]
    gx = gbb[:, 0:1]
    gy = gbb[:, 1:2]
    gw = gbb[:, 2:3]
    gh = gbb[:, 3:4]
    gxe = gx + gw - 1.0
    gye = gy + gh - 1.0
    cx = gx + gw / 2.0
    cy = gy + gh / 2.0
    meas = jnp.maximum(gw, gh)

    d1 = px - gx
    d2 = py - gy
    d3 = gxe - px
    d4 = gye - py
    hit = jnp.minimum(jnp.minimum(d1, d2), jnp.minimum(d3, d4)) >= 0.0
    q = (jnp.maximum(jnp.minimum(d1, d3), 0.0) /
         jnp.maximum(jnp.maximum(d1, d3), 0.01)) * \
        (jnp.maximum(jnp.minimum(d2, d4), 0.0) /
         jnp.maximum(jnp.maximum(d2, d4), 0.01))
    q = jnp.sqrt(q)

    inside_core = (px >= cx - s2) & (px <= cx + s2) & \
                  (py >= cy - s2) & (py <= cy + s2) & hit
    q = jnp.where(inside_core, 1.0, q)

    ghs = (rlo <= meas) & (meas <= rhi)
    green = ghs & hit
    gray = (glo <= meas) & (meas <= ghi) & ~ghs & hit
    q = q * green.astype(f32)

    G = LG // 2
    qA = q[0:G, :]
    qB = q[G:LG, :]
    maxA = jnp.max(qA, axis=0, keepdims=True)
    maxB = jnp.max(qB, axis=0, keepdims=True)
    gidx = jax.lax.broadcasted_iota(jnp.int32, (G, NL), 0)
    candA = jnp.where(qA == maxA, gidx, G)
    candB = jnp.where(qB == maxB, gidx, G)
    midxA = jnp.min(candA, axis=0, keepdims=True)
    midxB = jnp.min(candB, axis=0, keepdims=True)
    sel = jnp.concatenate(
        [(gidx == midxA), (gidx == midxB)], axis=0).astype(f32)

    gbr = gtt_ref[...]
    gx_r = gbr[0:1, :]
    gy_r = gbr[1:2, :]
    gxe_r = gx_r + gbr[2:3, :] - 1.0
    gye_r = gy_r + gbr[3:4, :] - 1.0
    lab_r = labr_ref[0].astype(f32)
    half_r = jax.lax.broadcasted_iota(jnp.int32, (1, LG), 1) >= G
    m5 = jnp.concatenate([gx_r, gy_r, gxe_r, gye_r, lab_r], axis=0)
    z3 = jnp.zeros((3, LG), f32)
    gm = jnp.concatenate(
        [jnp.where(half_r, 0.0, m5), z3, jnp.where(half_r, m5, 0.0), z3],
        axis=0)
    target_r = labr_ref[0] + jnp.where(half_r, 128, 0)
    oh = (jax.lax.broadcasted_iota(jnp.int32, (256, LG), 0) ==
          target_r).astype(f32)

    t = jnp.dot(gm, sel, preferred_element_type=f32,
                precision=jax.lax.Precision.HIGHEST)
    gcv = jnp.dot(oh, gray.astype(f32), preferred_element_type=f32)

    ciota = jax.lax.broadcasted_iota(jnp.int32, (NUM_CLASSES, NL), 0)

    posA = maxA > 0.0
    matchedA = t[4:5, :].astype(jnp.int32)
    grayA = gcv[0:NUM_CLASSES, :] > 0.0
    clsA = jnp.where((ciota == matchedA) & posA, maxA,
                     jnp.where(grayA, -1.0, 0.0))
    cls_ref[0] = clsA
    regA = jnp.concatenate(
        [px - t[0:1, :], py - t[1:2, :], t[2:3, :] - px, t[3:4, :] - py],
        axis=0) * posA.astype(f32)
    reg_ref[0] = regA

    posB = maxB > 0.0
    matchedB = t[12:13, :].astype(jnp.int32)
    grayB = gcv[128:128 + NUM_CLASSES, :] > 0.0
    clsB = jnp.where((ciota == matchedB) & posB, maxB,
                     jnp.where(grayB, -1.0, 0.0))
    cls_ref[1] = clsB
    regB = jnp.concatenate(
        [px - t[8:9, :], py - t[9:10, :], t[10:11, :] - px, t[11:12, :] - py],
        axis=0) * posB.astype(f32)
    reg_ref[1] = regB


def kernel(points, reg_ranges, gray_ranges, strides, gt_bboxes, gt_labels):
    P = points.shape[0]
    B, G, _ = gt_bboxes.shape
    assert PPAD % NL == 0 and B % 2 == 0 and 2 * G == LG
    npb = PPAD // NL
    B2 = B // 2

    ptall = jnp.concatenate(
        [points, reg_ranges, gray_ranges, strides.reshape(P, 1)], axis=1)
    pt_t = jnp.pad(ptall, ((0, PPAD - P), (0, 0))).T
    gt_raw = gt_bboxes.reshape(B * G, 4)
    gt_t = gt_bboxes.reshape(B * G, 4).T
    lab_row = gt_labels.reshape(B2, 1, LG)

    cls_t, reg_t = pl.pallas_call(
        _assign_kernel,
        grid=(B2, npb),
        in_specs=[
            pl.BlockSpec((7, NL), lambda b, i: (0, i)),
            pl.BlockSpec((LG, 4), lambda b, i: (b, 0)),
            pl.BlockSpec((4, LG), lambda b, i: (0, b)),
            pl.BlockSpec((1, 1, LG), lambda b, i: (b, 0, 0)),
        ],
        out_specs=[
            pl.BlockSpec((2, NUM_CLASSES, NL), lambda b, i: (b, 0, i)),
            pl.BlockSpec((2, 4, NL), lambda b, i: (b, 0, i)),
        ],
        out_shape=[
            jax.ShapeDtypeStruct((B, NUM_CLASSES, PPAD), jnp.float32),
            jax.ShapeDtypeStruct((B, 4, PPAD), jnp.float32),
        ],
        compiler_params=pltpu.CompilerParams(
            dimension_semantics=("parallel", "arbitrary"),
        ),
    )(pt_t, gt_raw, gt_t, lab_row)
    cls = jnp.transpose(cls_t[:, :, :P], (0, 2, 1))
    reg = jnp.transpose(reg_t[:, :, :P], (0, 2, 1))
    return cls, reg
